# Initial kernel scaffold; baseline (speedup 1.0000x reference)
import functools

import jax
import jax.numpy as jnp
import numpy as np
from jax import lax
from jax.experimental import pallas as pl
from jax.experimental.pallas import tpu as pltpu
from jax.experimental.pallas import tpu_sc as plsc

OUT_HW = 7
SR = 2
SCALE = 0.25
P = OUT_HW * OUT_HW * SR * SR * 4
NC, NS = 2, 16
NW = NC * NS
GCHUNK = 112
NCHUNK = P // GCHUNK


def _slot_constants():
    p = np.arange(P)
    b = p // 16
    s = (p % 16) // 4
    corner = p % 4
    iy, jx = b // OUT_HW, b % OUT_HW
    si, sj = s // SR, s % SR
    gy = iy + (si + 0.5) / SR
    gx = jx + (sj + 0.5) / SR
    dy, dx = corner // 2, corner % 2
    return (gy.astype(np.float32), gx.astype(np.float32),
            dy.astype(np.float32), dx.astype(np.float32))


GY, GX, DY, DX = _slot_constants()


def _prep_pallas(rois_pad, H, W, interpret=False):
    KP = rois_pad.shape[0]
    KB = 128
    gy = jnp.asarray(GY)[None, :]
    gx = jnp.asarray(GX)[None, :]
    dyb = jnp.asarray(DY > 0.5)[None, :]
    dxb = jnp.asarray(DX > 0.5)[None, :]
    dyf = jnp.asarray(DY)[None, :]
    dxf = jnp.asarray(DX)[None, :]

    def body(r_ref, idx_ref, w_ref):
        r = r_ref[...]
        sw = r[:, 1:2] * SCALE
        sh = r[:, 2:3] * SCALE
        ew = r[:, 3:4] * SCALE
        eh = r[:, 4:5] * SCALE
        bh = jnp.maximum(eh - sh, 1.0) / OUT_HW
        bw = jnp.maximum(ew - sw, 1.0) / OUT_HW
        ys = sh + gy * bh
        xs = sw + gx * bw
        valid = ((ys >= -1.0) & (ys <= H * 1.0) &
                 (xs >= -1.0) & (xs <= W * 1.0))
        y = jnp.clip(ys, 0.0, H - 1.0)
        x = jnp.clip(xs, 0.0, W - 1.0)
        y0 = jnp.clip(jnp.floor(y), 0.0, H - 2.0)
        x0 = jnp.clip(jnp.floor(x), 0.0, W - 2.0)
        ly = y - y0
        lx = x - x0
        wy = jnp.where(dyb, ly, 1.0 - ly)
        wx = jnp.where(dxb, lx, 1.0 - lx)
        w_ref[...] = jnp.where(valid, wy * wx * (1.0 / (SR * SR)), 0.0)
        idx_ref[...] = ((y0 + dyf) * W + (x0 + dxf)).astype(jnp.int32)

    return pl.pallas_call(
        body,
        grid=(KP // KB,),
        in_specs=[pl.BlockSpec((KB, 5), lambda i: (i, 0))],
        out_specs=[pl.BlockSpec((KB, P), lambda i: (i, 0)),
                   pl.BlockSpec((KB, P), lambda i: (i, 0))],
        out_shape=[jax.ShapeDtypeStruct((KP, P), jnp.int32),
                   jax.ShapeDtypeStruct((KP, P), jnp.float32)],
        interpret=interpret,
    )(rois_pad)


def _transpose_pallas(feature, interpret=False):
    _, C, H, W = feature.shape

    def body(f_ref, o_ref):
        x = f_ref[...].reshape(C, W)
        o_ref[...] = jnp.transpose(x, (1, 0))[None]

    return pl.pallas_call(
        body,
        grid=(H,),
        in_specs=[pl.BlockSpec((1, C, 1, W), lambda h: (0, 0, h, 0))],
        out_specs=pl.BlockSpec((1, W, C), lambda h: (h, 0, 0)),
        out_shape=jax.ShapeDtypeStruct((H, W, C), jnp.float32),
        interpret=interpret,
    )(feature)


def _sc_roialign(feat2d, idxs, ws):
    HW, C = feat2d.shape
    KP = idxs.shape[0]
    RPT = KP // NW
    NV = C // 16
    NB = OUT_HW * OUT_HW
    mesh = plsc.VectorSubcoreMesh(core_axis_name="c", subcore_axis_name="s")

    @functools.partial(
        pl.kernel,
        out_type=jax.ShapeDtypeStruct((KP, C, NB), jnp.float32),
        mesh=mesh,
        scratch_types=[
            pltpu.VMEM((NCHUNK, GCHUNK), jnp.int32),
            pltpu.VMEM((P,), jnp.float32),
            pltpu.VMEM((P, C), jnp.float32),
            pltpu.VMEM((C, NB), jnp.float32),
            pltpu.SemaphoreType.DMA,
        ],
    )
    def k(feat_hbm, idx_hbm, w_hbm, out_hbm, idx_v, w_v, rows_v, slab_v, gsem):
        wid = lax.axis_index("s") * NC + lax.axis_index("c")
        base = wid * RPT
        lanes = lax.iota(jnp.int32, 16)

        def roi_body(r, carry):
            kk = base + r
            pltpu.sync_copy(idx_hbm.at[kk], idx_v)
            pltpu.sync_copy(w_hbm.at[kk], w_v)
            descs = [
                pltpu.async_copy(feat_hbm.at[idx_v.at[c]],
                                 rows_v.at[pl.ds(c * GCHUNK, GCHUNK)], gsem)
                for c in range(NCHUNK)
            ]
            for d in descs:
                d.wait()

            def bin_body(b, c2):
                accs = [jnp.zeros((16,), jnp.float32) for _ in range(NV)]
                for j in range(16):
                    row = b * 16 + j
                    wj = plsc.load_gather(
                        w_v, [jnp.full((16,), row, dtype=jnp.int32)])
                    for v in range(NV):
                        accs[v] = accs[v] + wj * rows_v[row, pl.ds(v * 16, 16)]
                colb = jnp.full((16,), b, dtype=jnp.int32)
                for v in range(NV):
                    plsc.store_scatter(slab_v, [lanes + v * 16, colb], accs[v])
                return c2

            lax.fori_loop(0, NB, bin_body, 0)
            pltpu.sync_copy(slab_v, out_hbm.at[kk])
            return carry

        lax.fori_loop(0, RPT, roi_body, 0)

    return k(feat2d, idxs, ws)


def kernel(feature, rois):
    N, C, H, W = feature.shape
    K = rois.shape[0]
    KP = ((K + 128 * NW - 1) // (128 * NW)) * (128 * NW)
    rois_pad = jnp.pad(rois, ((0, KP - K), (0, 0)))
    idx, w = _prep_pallas(rois_pad, H, W)
    feat_hwc = _transpose_pallas(feature)
    feat2d = feat_hwc.reshape(H * W, C)
    out = _sc_roialign(feat2d, idx.reshape(KP, NCHUNK, GCHUNK), w)
    return out.reshape(KP, C, OUT_HW, OUT_HW)[:K]


# speedup vs baseline: 15.2245x; 15.2245x over previous
import functools

import jax
import jax.numpy as jnp
import numpy as np
from jax import lax
from jax.experimental import pallas as pl
from jax.experimental.pallas import tpu as pltpu
from jax.experimental.pallas import tpu_sc as plsc

OUT_HW = 7
SR = 2
SCALE = 0.25
P = OUT_HW * OUT_HW * SR * SR * 4
NC, NS = 2, 16
NW = NC * NS
GCHUNK = 112
NCHUNK = P // GCHUNK


def _slot_constants():
    p = np.arange(P)
    b = p // 16
    s = (p % 16) // 4
    corner = p % 4
    iy, jx = b // OUT_HW, b % OUT_HW
    si, sj = s // SR, s % SR
    gy = iy + (si + 0.5) / SR
    gx = jx + (sj + 0.5) / SR
    dy, dx = corner // 2, corner % 2
    return (gy.astype(np.float32), gx.astype(np.float32),
            dy.astype(np.float32), dx.astype(np.float32))


GY, GX, DY, DX = _slot_constants()


def _prep_pallas(rois_pad, H, W, interpret=False):
    KP = rois_pad.shape[0]
    KB = 128
    consts = jnp.asarray(np.stack([GY, GX, DY, DX]))

    def body(r_ref, c_ref, idx_ref, w_ref):
        r = r_ref[...]
        gy = c_ref[0:1, :]
        gx = c_ref[1:2, :]
        dyf = c_ref[2:3, :]
        dxf = c_ref[3:4, :]
        dyb = dyf > 0.5
        dxb = dxf > 0.5
        sw = r[:, 1:2] * SCALE
        sh = r[:, 2:3] * SCALE
        ew = r[:, 3:4] * SCALE
        eh = r[:, 4:5] * SCALE
        bh = jnp.maximum(eh - sh, 1.0) / OUT_HW
        bw = jnp.maximum(ew - sw, 1.0) / OUT_HW
        ys = sh + gy * bh
        xs = sw + gx * bw
        valid = ((ys >= -1.0) & (ys <= H * 1.0) &
                 (xs >= -1.0) & (xs <= W * 1.0))
        y = jnp.clip(ys, 0.0, H - 1.0)
        x = jnp.clip(xs, 0.0, W - 1.0)
        y0 = jnp.clip(jnp.floor(y), 0.0, H - 2.0)
        x0 = jnp.clip(jnp.floor(x), 0.0, W - 2.0)
        ly = y - y0
        lx = x - x0
        wy = jnp.where(dyb, ly, 1.0 - ly)
        wx = jnp.where(dxb, lx, 1.0 - lx)
        w_ref[...] = jnp.where(valid, wy * wx * (1.0 / (SR * SR)), 0.0)
        idx_ref[...] = ((y0 + dyf) * W + (x0 + dxf)).astype(jnp.int32)

    return pl.pallas_call(
        body,
        grid=(KP // KB,),
        in_specs=[pl.BlockSpec((KB, 5), lambda i: (i, 0)),
                  pl.BlockSpec((4, P), lambda i: (0, 0))],
        out_specs=[pl.BlockSpec((KB, P), lambda i: (i, 0)),
                   pl.BlockSpec((KB, P), lambda i: (i, 0))],
        out_shape=[jax.ShapeDtypeStruct((KP, P), jnp.int32),
                   jax.ShapeDtypeStruct((KP, P), jnp.float32)],
        interpret=interpret,
    )(rois_pad, consts)


def _transpose_pallas(feature, interpret=False):
    _, C, H, W = feature.shape

    HB = 8

    def body(f_ref, o_ref):
        for hh in range(HB):
            o_ref[hh, :, :] = jnp.transpose(f_ref[0, :, hh, :], (1, 0))

    return pl.pallas_call(
        body,
        grid=(H // HB,),
        in_specs=[pl.BlockSpec((1, C, HB, W), lambda h: (0, 0, h, 0))],
        out_specs=pl.BlockSpec((HB, W, C), lambda h: (h, 0, 0)),
        out_shape=jax.ShapeDtypeStruct((H, W, C), jnp.float32),
        interpret=interpret,
    )(feature)


def _sc_roialign(feat2d, idxs, ws):
    HW, C = feat2d.shape
    KP = idxs.shape[0]
    RPT = KP // NW
    NV = C // 16
    NB = OUT_HW * OUT_HW
    mesh = plsc.VectorSubcoreMesh(core_axis_name="c", subcore_axis_name="s")

    @functools.partial(
        pl.kernel,
        out_type=jax.ShapeDtypeStruct((KP, C, NB), jnp.float32),
        mesh=mesh,
        scratch_types=[
            pltpu.VMEM((NCHUNK, GCHUNK), jnp.int32),
            pltpu.VMEM((P,), jnp.float32),
            pltpu.VMEM((P, C), jnp.float32),
            pltpu.VMEM((C, NB), jnp.float32),
            pltpu.SemaphoreType.DMA,
        ],
        compiler_params=pltpu.CompilerParams(needs_layout_passes=False),
    )
    def k(feat_hbm, idx_hbm, w_hbm, out_hbm, idx_v, w_v, rows_v, slab_v, gsem):
        wid = lax.axis_index("s") * NC + lax.axis_index("c")
        base = wid * RPT
        lanes = lax.iota(jnp.int32, 16)

        def roi_body(r, carry):
            kk = base + r
            pltpu.sync_copy(idx_hbm.at[kk], idx_v)
            pltpu.sync_copy(w_hbm.at[kk], w_v)
            descs = [
                pltpu.async_copy(feat_hbm.at[idx_v.at[c]],
                                 rows_v.at[pl.ds(c * GCHUNK, GCHUNK)], gsem)
                for c in range(NCHUNK)
            ]
            for d in descs:
                d.wait()

            def bin_body(b, c2):
                accs = [jnp.zeros((16,), jnp.float32) for _ in range(NV)]
                for j in range(16):
                    row = b * 16 + j
                    wj = plsc.load_gather(
                        w_v, [jnp.full((16,), row, dtype=jnp.int32)])
                    for v in range(NV):
                        accs[v] = accs[v] + wj * rows_v[row, pl.ds(v * 16, 16)]
                colb = jnp.full((16,), b, dtype=jnp.int32)
                for v in range(NV):
                    plsc.store_scatter(slab_v, [lanes + v * 16, colb], accs[v])
                return c2

            lax.fori_loop(0, NB, bin_body, 0)
            pltpu.sync_copy(slab_v, out_hbm.at[kk])
            return carry

        lax.fori_loop(0, RPT, roi_body, 0)

    return k(feat2d, idxs, ws)


def kernel(feature, rois):
    N, C, H, W = feature.shape
    K = rois.shape[0]
    KP = ((K + 127) // 128) * 128
    rois_pad = jnp.pad(rois, ((0, KP - K), (0, 0)))
    idx, w = _prep_pallas(rois_pad, H, W)
    feat_hwc = _transpose_pallas(feature)
    feat2d = feat_hwc.reshape(H * W, C)
    out = _sc_roialign(feat2d, idx.reshape(KP, NCHUNK, GCHUNK), w)
    return out.reshape(KP, C, OUT_HW, OUT_HW)[:K]


# speedup vs baseline: 21.0401x; 1.3820x over previous
import functools

import jax
import jax.numpy as jnp
import numpy as np
from jax import lax
from jax.experimental import pallas as pl
from jax.experimental.pallas import tpu as pltpu
from jax.experimental.pallas import tpu_sc as plsc

OUT_HW = 7
SR = 2
SCALE = 0.25
P = OUT_HW * OUT_HW * SR * SR * 4
Q = P // 2
NC, NS = 2, 16
NW = NC * NS
GCHUNK = 56
NCHUNK = Q // GCHUNK


def _slot_constants():
    p = np.arange(P)
    q = p // 2
    b = q // 8
    s = (q % 8) // 2
    dx = q % 2
    dy = p % 2
    iy, jx = b // OUT_HW, b % OUT_HW
    si, sj = s // SR, s % SR
    gy = iy + (si + 0.5) / SR
    gx = jx + (sj + 0.5) / SR
    return (gy.astype(np.float32), gx.astype(np.float32),
            dy.astype(np.float32), dx.astype(np.float32))


GY, GX, DY, DX = _slot_constants()


def _prep_pallas(rois_pad, H, W, interpret=False):
    KP = rois_pad.shape[0]
    KB = 128
    consts = jnp.asarray(np.stack([GY, GX, DY, DX]))

    def body(r_ref, c_ref, idx_ref, w_ref):
        r = r_ref[...]
        gy = c_ref[0:1, :]
        gx = c_ref[1:2, :]
        dyf = c_ref[2:3, :]
        dxf = c_ref[3:4, :]
        dyb = dyf > 0.5
        dxb = dxf > 0.5
        sw = r[:, 1:2] * SCALE
        sh = r[:, 2:3] * SCALE
        ew = r[:, 3:4] * SCALE
        eh = r[:, 4:5] * SCALE
        bh = jnp.maximum(eh - sh, 1.0) / OUT_HW
        bw = jnp.maximum(ew - sw, 1.0) / OUT_HW
        ys = sh + gy * bh
        xs = sw + gx * bw
        valid = ((ys >= -1.0) & (ys <= H * 1.0) &
                 (xs >= -1.0) & (xs <= W * 1.0))
        y = jnp.clip(ys, 0.0, H - 1.0)
        x = jnp.clip(xs, 0.0, W - 1.0)
        y0 = jnp.clip(jnp.floor(y), 0.0, H - 2.0)
        x0 = jnp.clip(jnp.floor(x), 0.0, W - 2.0)
        ly = y - y0
        lx = x - x0
        wy = jnp.where(dyb, ly, 1.0 - ly)
        wx = jnp.where(dxb, lx, 1.0 - lx)
        w_ref[...] = jnp.where(valid, wy * wx * (1.0 / (SR * SR)), 0.0)
        idx_ref[...] = (y0 * W + x0 + dxf).astype(jnp.int32)

    idx2, w = pl.pallas_call(
        body,
        grid=(KP // KB,),
        in_specs=[pl.BlockSpec((KB, 5), lambda i: (i, 0)),
                  pl.BlockSpec((4, P), lambda i: (0, 0))],
        out_specs=[pl.BlockSpec((KB, P), lambda i: (i, 0)),
                   pl.BlockSpec((KB, P), lambda i: (i, 0))],
        out_shape=[jax.ShapeDtypeStruct((KP, P), jnp.int32),
                   jax.ShapeDtypeStruct((KP, P), jnp.float32)],
        interpret=interpret,
    )(rois_pad, consts)
    idx = idx2.reshape(KP, Q, 2)[:, :, 0]
    return idx, w


def _transpose_pallas(feature, interpret=False):
    _, C, H, W = feature.shape

    HB = 8

    def body(f_ref, o_ref):
        for hh in range(HB):
            x = jnp.transpose(f_ref[0, :, hh, :], (1, 0))
            o_ref[hh, :, :] = x.astype(jnp.bfloat16)

    return pl.pallas_call(
        body,
        grid=(H // HB,),
        in_specs=[pl.BlockSpec((1, C, HB, W), lambda h: (0, 0, h, 0))],
        out_specs=pl.BlockSpec((HB, W, C), lambda h: (h, 0, 0)),
        out_shape=jax.ShapeDtypeStruct((H, W, C), jnp.bfloat16),
        interpret=interpret,
    )(feature)


def _sc_roialign(feat_i32, idxs, ws, C):
    KP = idxs.shape[0]
    RPT = KP // NW
    NB = OUT_HW * OUT_HW
    G = 8
    NG = RPT // G
    NWORD = C // 32
    HALF = C // 2
    NCA, NCB = 4, 3
    BA = NCA * GCHUNK // 8
    BB = NCB * GCHUNK // 8
    mesh = plsc.VectorSubcoreMesh(core_axis_name="c", subcore_axis_name="s")

    @functools.partial(
        pl.kernel,
        out_type=jax.ShapeDtypeStruct((KP, C, NB), jnp.float32),
        mesh=mesh,
        scratch_types=[
            pltpu.VMEM((G, NCHUNK, GCHUNK), jnp.int32),
            pltpu.VMEM((G * P,), jnp.float32),
            pltpu.VMEM((NCA * GCHUNK, C), jnp.int32),
            pltpu.VMEM((NCB * GCHUNK, C), jnp.int32),
            pltpu.VMEM((C, NB), jnp.float32),
            pltpu.SemaphoreType.DMA,
            pltpu.SemaphoreType.DMA,
        ],
        compiler_params=pltpu.CompilerParams(needs_layout_passes=False),
    )
    def k(feat_hbm, idx_hbm, w_hbm, out_hbm,
          idx_g, w_g, buf_a, buf_b, slab_v, sema, semb):
        wid = lax.axis_index("s") * NC + lax.axis_index("c")
        base = wid * RPT
        lanes = lax.iota(jnp.int32, 16)

        def issue(rr, c0, nch, buf, sem):
            return [
                pltpu.async_copy(feat_hbm.at[idx_g.at[rr, c0 + c]],
                                 buf.at[pl.ds(c * GCHUNK, GCHUNK)], sem)
                for c in range(nch)
            ]

        def compute(rr, buf, b0, nbins):
            def bin_body(bl, c2):
                b = b0 + bl
                accs = [jnp.zeros((16,), jnp.float32) for _ in range(2 * NWORD)]
                for m in range(8):
                    row = bl * 8 + m
                    wbase = rr * P + b * 16 + m * 2
                    wt = plsc.load_gather(
                        w_g, [jnp.full((16,), wbase, dtype=jnp.int32)])
                    wb = plsc.load_gather(
                        w_g, [jnp.full((16,), wbase + 1, dtype=jnp.int32)])
                    for v in range(NWORD):
                        ptop = buf[row, pl.ds(v * 16, 16)]
                        ev, od = plsc.unpack(
                            plsc.bitcast(ptop, jnp.bfloat16),
                            format=plsc.PackFormat.INTERLEAVED)
                        accs[2 * v] = accs[2 * v] + wt * ev
                        accs[2 * v + 1] = accs[2 * v + 1] + wt * od
                        pbot = buf[row, pl.ds(HALF + v * 16, 16)]
                        ev2, od2 = plsc.unpack(
                            plsc.bitcast(pbot, jnp.bfloat16),
                            format=plsc.PackFormat.INTERLEAVED)
                        accs[2 * v] = accs[2 * v] + wb * ev2
                        accs[2 * v + 1] = accs[2 * v + 1] + wb * od2
                colb = jnp.full((16,), b, dtype=jnp.int32)
                for v in range(NWORD):
                    plsc.store_scatter(
                        slab_v, [v * 32 + 2 * lanes, colb], accs[2 * v])
                    plsc.store_scatter(
                        slab_v, [v * 32 + 2 * lanes + 1, colb], accs[2 * v + 1])
                return c2

            lax.fori_loop(0, nbins, bin_body, 0)

        def group_body(g, carry):
            k0 = base + g * G
            pltpu.sync_copy(idx_hbm.at[pl.ds(k0, G)], idx_g)
            pltpu.sync_copy(w_hbm.at[wid * NG + g], w_g)
            da = issue(0, 0, NCA, buf_a, sema)
            for rr in range(G):
                for d in da:
                    d.wait()
                db = issue(rr, NCA, NCB, buf_b, semb)
                compute(rr, buf_a, 0, BA)
                for d in db:
                    d.wait()
                if rr + 1 < G:
                    da = issue(rr + 1, 0, NCA, buf_a, sema)
                else:
                    da = []
                compute(rr, buf_b, BA, BB)
                pltpu.sync_copy(slab_v, out_hbm.at[k0 + rr])
            return carry

        lax.fori_loop(0, NG, group_body, 0)

    return k(feat_i32, idxs, ws.reshape(KP // G, G * P))


def kernel(feature, rois):
    N, C, H, W = feature.shape
    K = rois.shape[0]
    KP = ((K + 127) // 128) * 128
    rois_pad = jnp.pad(rois, ((0, KP - K), (0, 0)))
    idx, w = _prep_pallas(rois_pad, H, W)
    fb = _transpose_pallas(feature)
    fb_dn = jnp.concatenate([fb[1:], fb[-1:]], axis=0)
    pair = jnp.concatenate([fb, fb_dn], axis=-1)
    feat_i32 = lax.bitcast_convert_type(
        pair.reshape(H * W, C, 2), jnp.int32)
    out = _sc_roialign(feat_i32, idx.reshape(KP, NCHUNK, GCHUNK), w, C)
    return out.reshape(KP, C, OUT_HW, OUT_HW)[:K]


# speedup vs baseline: 24.3336x; 1.1565x over previous
import functools

import jax
import jax.numpy as jnp
import numpy as np
from jax import lax
from jax.experimental import pallas as pl
from jax.experimental.pallas import tpu as pltpu
from jax.experimental.pallas import tpu_sc as plsc

OUT_HW = 7
SR = 2
SCALE = 0.25
P = OUT_HW * OUT_HW * SR * SR * 4
Q = P // 2
NC, NS = 2, 16
NW = NC * NS
NCI = 4
QP = NCI * 128
PP = 896
QTAIL = Q - 3 * 128


def _slot_constants():
    p = np.arange(P)
    q = p // 2
    b = q // 8
    s = (q % 8) // 2
    dx = q % 2
    dy = p % 2
    iy, jx = b // OUT_HW, b % OUT_HW
    si, sj = s // SR, s % SR
    gy = iy + (si + 0.5) / SR
    gx = jx + (sj + 0.5) / SR
    return (gy.astype(np.float32), gx.astype(np.float32),
            dy.astype(np.float32), dx.astype(np.float32))


def _gather_slot_constants():
    q = np.arange(Q)
    b = q // 8
    s = (q % 8) // 2
    dx = q % 2
    iy, jx = b // OUT_HW, b % OUT_HW
    si, sj = s // SR, s % SR
    gy = iy + (si + 0.5) / SR
    gx = jx + (sj + 0.5) / SR
    return (gy.astype(np.float32), gx.astype(np.float32),
            dx.astype(np.float32), np.zeros(Q, np.float32))


GY, GX, DY, DX = _slot_constants()
GYQ, GXQ, DXQ, _ZQ = _gather_slot_constants()


def _prep_pallas(rois_pad, H, W, interpret=False):
    KP = rois_pad.shape[0]
    KB = 128
    consts = jnp.asarray(np.stack([GY, GX, DY, DX]))
    qpad = np.zeros((3, QP), np.float32)
    qpad[0, :Q] = GYQ
    qpad[1, :Q] = GXQ
    qpad[2, :Q] = DXQ
    constsq = jnp.asarray(qpad.reshape(3, NCI, 128))

    def body(r_ref, c_ref, cq_ref, idx_ref, w_ref):
        r = r_ref[...]
        gy = c_ref[0:1, :]
        gx = c_ref[1:2, :]
        dyf = c_ref[2:3, :]
        dxf = c_ref[3:4, :]
        dyb = dyf > 0.5
        dxb = dxf > 0.5
        sw = r[:, 1:2] * SCALE
        sh = r[:, 2:3] * SCALE
        ew = r[:, 3:4] * SCALE
        eh = r[:, 4:5] * SCALE
        bh = jnp.maximum(eh - sh, 1.0) / OUT_HW
        bw = jnp.maximum(ew - sw, 1.0) / OUT_HW
        ys = sh + gy * bh
        xs = sw + gx * bw
        valid = ((ys >= -1.0) & (ys <= H * 1.0) &
                 (xs >= -1.0) & (xs <= W * 1.0))
        y = jnp.clip(ys, 0.0, H - 1.0)
        x = jnp.clip(xs, 0.0, W - 1.0)
        y0 = jnp.clip(jnp.floor(y), 0.0, H - 2.0)
        x0 = jnp.clip(jnp.floor(x), 0.0, W - 2.0)
        ly = y - y0
        lx = x - x0
        wy = jnp.where(dyb, ly, 1.0 - ly)
        wx = jnp.where(dxb, lx, 1.0 - lx)
        wv = jnp.where(valid, wy * wx * (1.0 / (SR * SR)), 0.0)
        w_ref[...] = jnp.concatenate(
            [wv, jnp.zeros((KB, PP - P), jnp.float32)], axis=1)
        gyq = cq_ref[0][None]
        gxq = cq_ref[1][None]
        dxq = cq_ref[2][None]
        sh3 = sh[:, :, None]
        sw3 = sw[:, :, None]
        bh3 = bh[:, :, None]
        bw3 = bw[:, :, None]
        ysq = sh3 + gyq * bh3
        xsq = sw3 + gxq * bw3
        y0q = jnp.clip(jnp.floor(jnp.clip(ysq, 0.0, H - 1.0)), 0.0, H - 2.0)
        x0q = jnp.clip(jnp.floor(jnp.clip(xsq, 0.0, W - 1.0)), 0.0, W - 2.0)
        idx_ref[...] = (y0q * W + x0q + dxq).astype(jnp.int32)

    idx, w = pl.pallas_call(
        body,
        grid=(KP // KB,),
        in_specs=[pl.BlockSpec((KB, 5), lambda i: (i, 0)),
                  pl.BlockSpec((4, P), lambda i: (0, 0)),
                  pl.BlockSpec((3, NCI, 128), lambda i: (0, 0, 0))],
        out_specs=[pl.BlockSpec((KB, NCI, 128), lambda i: (i, 0, 0)),
                   pl.BlockSpec((KB, PP), lambda i: (i, 0))],
        out_shape=[jax.ShapeDtypeStruct((KP, NCI, 128), jnp.int32),
                   jax.ShapeDtypeStruct((KP, PP), jnp.float32)],
        interpret=interpret,
    )(rois_pad, consts, constsq)
    return idx, w


def _pack_pallas(feature, interpret=False):
    _, C, H, W = feature.shape
    HB = 8
    NBLK = H // HB

    def body(f_ref, g_ref, o_ref):
        for hh in range(HB):
            top = f_ref[0, :, hh, :]
            bot = f_ref[0, :, hh + 1, :] if hh + 1 < HB else g_ref[0, :, 0, :]
            tt = jnp.transpose(top, (1, 0)).astype(jnp.bfloat16)
            bb = jnp.transpose(bot, (1, 0)).astype(jnp.bfloat16)
            ti = lax.bitcast_convert_type(tt, jnp.uint16).astype(jnp.int32)
            bi = lax.bitcast_convert_type(bb, jnp.uint16).astype(jnp.int32)
            o_ref[hh, :, :] = ti | (bi << 16)

    return pl.pallas_call(
        body,
        grid=(NBLK,),
        in_specs=[
            pl.BlockSpec((1, C, HB, W), lambda h: (0, 0, h, 0)),
            pl.BlockSpec((1, C, HB, W),
                         lambda h: (0, 0, jnp.minimum(h + 1, NBLK - 1), 0)),
        ],
        out_specs=pl.BlockSpec((HB, W, C), lambda h: (h, 0, 0)),
        out_shape=jax.ShapeDtypeStruct((H, W, C), jnp.int32),
        interpret=interpret,
    )(feature, feature)


def _sc_roialign(feat_pk, idxs, ws, C, K):
    KP = idxs.shape[0]
    RPT = KP // NW
    NB = OUT_HW * OUT_HW
    G = 8
    NG = RPT // G
    NWORD = C // 16
    RA = 256
    RB = Q - RA
    BA = RA // 8
    BB = RB // 8
    mesh = plsc.VectorSubcoreMesh(core_axis_name="c", subcore_axis_name="s")

    @functools.partial(
        pl.kernel,
        out_type=jax.ShapeDtypeStruct((K, C, NB), jnp.float32),
        mesh=mesh,
        scratch_types=[
            pltpu.VMEM((G, NCI, 128), jnp.int32),
            pltpu.VMEM((G * PP,), jnp.float32),
            pltpu.VMEM((RA, C), jnp.int32),
            pltpu.VMEM((RB, C), jnp.int32),
            pltpu.VMEM((C, NB), jnp.float32),
            pltpu.SemaphoreType.DMA,
            pltpu.SemaphoreType.DMA,
            pltpu.SemaphoreType.DMA,
        ],
        compiler_params=pltpu.CompilerParams(needs_layout_passes=False),
    )
    def k(feat_hbm, idx_hbm, w_hbm, out_hbm,
          idx_g, w_g, buf_a, buf_b, slab_v, sema, semb, semio):
        wid = lax.axis_index("s") * NC + lax.axis_index("c")
        base = wid * RPT
        lanes = lax.iota(jnp.int32, 16)

        def issue_a(rr):
            return [
                pltpu.async_copy(feat_hbm.at[idx_g.at[rr, 0]],
                                 buf_a.at[pl.ds(0, 128)], sema),
                pltpu.async_copy(feat_hbm.at[idx_g.at[rr, 1]],
                                 buf_a.at[pl.ds(128, 128)], sema),
            ]

        def issue_b(rr):
            return [
                pltpu.async_copy(feat_hbm.at[idx_g.at[rr, 2]],
                                 buf_b.at[pl.ds(0, 128)], semb),
                pltpu.async_copy(feat_hbm.at[idx_g.at[rr, 3, pl.ds(0, QTAIL)]],
                                 buf_b.at[pl.ds(128, QTAIL)], semb),
            ]

        def compute(rr, buf, b0, nbins):
            def bin_body(bl, c2):
                b = b0 + bl
                accs = [jnp.zeros((16,), jnp.float32) for _ in range(NWORD)]
                for m in range(8):
                    row = bl * 8 + m
                    wbase = rr * PP + b * 16 + m * 2
                    wt = plsc.load_gather(
                        w_g, [jnp.full((16,), wbase, dtype=jnp.int32)])
                    wb = plsc.load_gather(
                        w_g, [jnp.full((16,), wbase + 1, dtype=jnp.int32)])
                    for v in range(NWORD):
                        pk = buf[row, pl.ds(v * 16, 16)]
                        top, bot = plsc.unpack(
                            plsc.bitcast(pk, jnp.bfloat16),
                            format=plsc.PackFormat.INTERLEAVED)
                        accs[v] = accs[v] + wt * top + wb * bot
                colb = jnp.full((16,), b, dtype=jnp.int32)
                for v in range(NWORD):
                    plsc.store_scatter(
                        slab_v, [v * 16 + lanes, colb], accs[v])
                return c2

            lax.fori_loop(0, nbins, bin_body, 0)

        def group_body(g, carry):
            k0 = base + g * G
            dio = [pltpu.async_copy(idx_hbm.at[pl.ds(k0, G)], idx_g, semio)]
            dio += [
                pltpu.async_copy(w_hbm.at[k0 + rr],
                                 w_g.at[pl.ds(rr * PP, PP)], semio)
                for rr in range(G)
            ]
            for d in dio:
                d.wait()
            da = issue_a(0)
            for rr in range(G):
                for d in da:
                    d.wait()
                db = issue_b(rr)
                compute(rr, buf_a, 0, BA)
                for d in db:
                    d.wait()
                if rr + 1 < G:
                    da = issue_a(rr + 1)
                else:
                    da = []
                compute(rr, buf_b, BA, BB)

                @pl.when(k0 + rr < K)
                def _():
                    pltpu.sync_copy(slab_v, out_hbm.at[k0 + rr])
            return carry

        lax.fori_loop(0, NG, group_body, 0)

    return k(feat_pk, idxs, ws)


def kernel(feature, rois):
    N, C, H, W = feature.shape
    K = rois.shape[0]
    KP = ((K + 127) // 128) * 128
    rois_pad = jnp.pad(rois, ((0, KP - K), (0, 0)))
    idx, w = _prep_pallas(rois_pad, H, W)
    feat_pk = _pack_pallas(feature).reshape(H * W, C)
    out = _sc_roialign(feat_pk, idx, w, C, K)
    return out.reshape(K, C, OUT_HW, OUT_HW)
